# baseline (speedup 1.0000x reference)
import functools

import jax
import jax.numpy as jnp
from jax import lax
from jax.experimental import pallas as pl
from jax.experimental.pallas import tpu as pltpu
from jax.experimental.pallas import tpu_sc as plsc

N = 10000
E = 320000
D = 128

NC = 2
NS = 16
NW = NC * NS
EPW = E // NW
EPS = E // NS
DH = D // NC
CHUNK = 125
STRIP = 8
NSTRIP = EPS // (STRIP * CHUNK)
NPAD = 10112
RPW = NPAD // NS
WPAD = 128
G16 = EPW // 16

_MESH = plsc.VectorSubcoreMesh(core_axis_name="c", subcore_axis_name="s")


def _seg_body(weighted, *refs):
    if weighted:
        (h_hbm, sdw_hbm, w_hbm, zeros_hbm, out_hbm,
         sdw0, sdw1, wv0, wv1, r0, r1, r2, r3,
         sg0, sg1, sg2, sg3, ss0, ss1, ss2, ss3, si0, si1, acc) = refs
        wvb = (wv0, wv1)
    else:
        (h_hbm, sdw_hbm, zeros_hbm, out_hbm,
         sdw0, sdw1, r0, r1, r2, r3,
         sg0, sg1, sg2, sg3, ss0, ss1, ss2, ss3, si0, si1, acc) = refs
        w_hbm = None
        wvb = (None, None)
    sdwb = (sdw0, sdw1)
    rb = (r0, r1, r2, r3)
    sgb = (sg0, sg1, sg2, sg3)
    ssb = (ss0, ss1, ss2, ss3)
    sib = (si0, si1)

    c = lax.axis_index("c")
    s = lax.axis_index("s")

    def load_strip(t, p, sync):
        if sync:
            pltpu.sync_copy(sdw_hbm.at[c, s * NSTRIP + t], sdwb[p])
            if weighted:
                pltpu.sync_copy(w_hbm.at[s * NSTRIP + t], wvb[p])
        else:
            pltpu.async_copy(sdw_hbm.at[c, s * NSTRIP + t], sdwb[p], sib[p])
            if weighted:
                pltpu.async_copy(w_hbm.at[s * NSTRIP + t], wvb[p], sib[p])

    def wait_strip(p):
        pltpu.make_async_copy(sdw_hbm.at[0, 0], sdwb[p], sib[p]).wait()
        if weighted:
            pltpu.make_async_copy(w_hbm.at[0], wvb[p], sib[p]).wait()

    def fire_gather(tp, q, b):
        pltpu.async_copy(h_hbm.at[sdwb[tp].at[0, q]], rb[b], sgb[b])

    def wait_gather(b):
        pltpu.make_async_copy(h_hbm.at[sdwb[0].at[0, 0]], rb[b],
                              sgb[b]).wait()

    def fire_scatter(tp, q, b):
        pltpu.async_copy(rb[b], acc.at[sdwb[tp].at[1, q]], ssb[b], add=True)

    def wait_scatter(b):
        pltpu.make_async_copy(rb[b], acc.at[sdwb[0].at[1, 0]],
                              ssb[b]).wait()

    def scale(tp, q, b):
        if not weighted:
            return
        rows = rb[b]
        wvt = wvb[tp]
        qf = jnp.full((16,), q, jnp.int32)

        def scale_g16(g, carry2):
            base = g * 16
            wv16 = wvt[q, pl.ds(base, 16)]
            for lane in range(16):
                ws = wv16[lane]
                j = base + lane
                for k in range(DH // 16):
                    sl = (j, pl.ds(k * 16, 16))
                    rows[sl] = rows[sl] * ws
            return carry2

        lax.fori_loop(0, CHUNK // 16, scale_g16, 0)
        rbase = (CHUNK // 16) * 16
        wv16 = wvt[q, pl.ds(rbase, 16)]
        for lane in range(CHUNK - rbase):
            ws = wv16[lane]
            j = rbase + lane
            for k in range(DH // 16):
                sl = (j, pl.ds(k * 16, 16))
                rows[sl] = rows[sl] * ws

    def run_strip(t, tp, first, last):
        if first:
            fire_gather(tp, 0, 0)
            fire_gather(tp, 1, 1)
            fire_gather(tp, 2, 2)
        for q in range(STRIP):
            b = q % 4
            wait_gather(b)
            scale(tp, q, b)
            fire_scatter(tp, q, b)
            if not last and q == 1:
                load_strip(t + 1, 1 - tp, sync=False)
            nq = q + 3
            if nq < STRIP:
                if first and q == 0:
                    fire_gather(tp, nq, nq % 4)
                else:
                    wait_scatter(nq % 4)
                    fire_gather(tp, nq, nq % 4)
            elif not last:
                if q == STRIP - 3:
                    wait_strip(1 - tp)
                wait_scatter(nq % 4)
                fire_gather(1 - tp, nq - STRIP, nq % 4)

    pltpu.sync_copy(zeros_hbm.at[pl.ds(s * RPW, RPW)],
                    acc.at[pl.ds(s * RPW, RPW)])
    load_strip(0, 0, sync=True)
    plsc.subcore_barrier()

    run_strip(0, 0, first=True, last=False)

    def strip_pair(u, carry):
        t1 = 2 * u + 1
        run_strip(t1, 1, first=False, last=False)
        run_strip(t1 + 1, 0, first=False, last=False)
        return carry

    lax.fori_loop(0, (NSTRIP - 2) // 2, strip_pair, 0)

    run_strip(NSTRIP - 1, 1, first=False, last=True)

    for b in range(4):
        wait_scatter(b)
    plsc.subcore_barrier()
    pltpu.sync_copy(acc.at[pl.ds(s * RPW, RPW)],
                    out_hbm.at[c, pl.ds(s * RPW, RPW)])


def _make_seg(weighted):
    wscratch = [
        pltpu.VMEM((STRIP, WPAD), jnp.float32),
        pltpu.VMEM((STRIP, WPAD), jnp.float32),
    ] if weighted else []
    return pl.kernel(
        functools.partial(_seg_body, weighted),
        out_type=jax.ShapeDtypeStruct((NC, NPAD, DH), jnp.float32),
        mesh=_MESH,
        compiler_params=pltpu.CompilerParams(
            needs_layout_passes=False, use_tc_tiling_on_sc=False),
        scratch_types=[
            pltpu.VMEM((2, STRIP, CHUNK), jnp.int32),
            pltpu.VMEM((2, STRIP, CHUNK), jnp.int32),
        ] + wscratch + [
            pltpu.VMEM((CHUNK, DH), jnp.float32),
            pltpu.VMEM((CHUNK, DH), jnp.float32),
            pltpu.VMEM((CHUNK, DH), jnp.float32),
            pltpu.VMEM((CHUNK, DH), jnp.float32),
            pltpu.SemaphoreType.DMA,
            pltpu.SemaphoreType.DMA,
            pltpu.SemaphoreType.DMA,
            pltpu.SemaphoreType.DMA,
            pltpu.SemaphoreType.DMA,
            pltpu.SemaphoreType.DMA,
            pltpu.SemaphoreType.DMA,
            pltpu.SemaphoreType.DMA,
            pltpu.SemaphoreType.DMA,
            pltpu.SemaphoreType.DMA,
            pltpu.VMEM_SHARED((NPAD, DH), jnp.float32),
        ],
    )


_seg_unweighted = _make_seg(False)
_seg_weighted = _make_seg(True)


def _w_body(px_hbm, py_hbm, pz_hbm, src_hbm, dst_hbm, w_out,
            pxv, pyv, pzv, srcv, dstv, wv):
    c = lax.axis_index("c")
    s = lax.axis_index("s")
    wid = s * NC + c

    pltpu.sync_copy(px_hbm, pxv)
    pltpu.sync_copy(py_hbm, pyv)
    pltpu.sync_copy(pz_hbm, pzv)
    pltpu.sync_copy(src_hbm.at[wid], srcv)
    pltpu.sync_copy(dst_hbm.at[wid], dstv)

    def group(g, carry):
        sl = pl.ds(g * 16, 16)
        s16 = srcv[sl]
        d16 = dstv[sl]
        ddx = plsc.load_gather(pxv, [s16]) - plsc.load_gather(pxv, [d16])
        ddy = plsc.load_gather(pyv, [s16]) - plsc.load_gather(pyv, [d16])
        ddz = plsc.load_gather(pzv, [s16]) - plsc.load_gather(pzv, [d16])
        d2 = ddx * ddx + ddy * ddy + ddz * ddz
        wv[sl] = jnp.exp(-d2)
        return carry

    lax.fori_loop(0, G16, group, 0)
    pltpu.sync_copy(wv, w_out.at[wid])


_w_kernel = pl.kernel(
    _w_body,
    out_type=jax.ShapeDtypeStruct((NW, EPW), jnp.float32),
    mesh=_MESH,
    compiler_params=pltpu.CompilerParams(needs_layout_passes=False),
    scratch_types=[
        pltpu.VMEM((N,), jnp.float32),
        pltpu.VMEM((N,), jnp.float32),
        pltpu.VMEM((N,), jnp.float32),
        pltpu.VMEM((EPW,), jnp.int32),
        pltpu.VMEM((EPW,), jnp.int32),
        pltpu.VMEM((EPW,), jnp.float32),
    ],
)


def _mlp_one(h, p, w1, b1, w2, b2, relu_out, normalize):
    t = h + jnp.concatenate([p[0, :N, :], p[1, :N, :]], axis=1)
    u = jnp.maximum(
        jnp.dot(t, w1, preferred_element_type=jnp.float32) + b1, 0.0)
    v = jnp.dot(u, w2, preferred_element_type=jnp.float32) + b2
    if relu_out:
        v = jnp.maximum(v, 0.0)
    if normalize:
        mu = jnp.mean(v, axis=0, keepdims=True)
        var = jnp.sum((v - mu) * (v - mu), axis=0, keepdims=True) / (N - 1)
        v = (v - mu) * lax.rsqrt(var)
    return v


def _mlp2_body(ha_ref, pa_ref, w1a_ref, b1a_ref, w2a_ref, b2a_ref,
               hb_ref, pb_ref, w1b_ref, b1b_ref, w2b_ref, b2b_ref,
               oa_ref, ob_ref, *, relu_out, normalize):
    oa_ref[...] = _mlp_one(ha_ref[...], pa_ref[...], w1a_ref[...],
                           b1a_ref[...], w2a_ref[...], b2a_ref[...],
                           relu_out, normalize)
    ob_ref[...] = _mlp_one(hb_ref[...], pb_ref[...], w1b_ref[...],
                           b1b_ref[...], w2b_ref[...], b2b_ref[...],
                           relu_out, normalize)


def _mlp2(ha, pa, w1a, b1a, w2a, b2a, hb, pb, w1b, b1b, w2b, b2b,
          relu_out, normalize):
    return pl.pallas_call(
        functools.partial(_mlp2_body, relu_out=relu_out, normalize=normalize),
        out_shape=(jax.ShapeDtypeStruct((N, D), jnp.float32),
                   jax.ShapeDtypeStruct((N, D), jnp.float32)),
    )(ha, pa, w1a, b1a, w2a, b2a, hb, pb, w1b, b1b, w2b, b2b)


def _pack(src, dst):
    out = []
    for cc in range(NC):
        a = jnp.stack([2 * src + cc, dst])
        a = a.reshape(2, NS * NSTRIP, STRIP, CHUNK)
        out.append(jnp.transpose(a, (1, 0, 2, 3)))
    return jnp.stack(out)


def kernel(x, edge_index, pos, W1a, b1a, W2a, b2a, W1b, b1b, W2b, b2b):
    src = edge_index[0]
    dst = edge_index[1]
    srcf = src.reshape(NW, EPW)
    dstf = dst.reshape(NW, EPW)
    px = jnp.asarray(pos[:, 0])
    py = jnp.asarray(pos[:, 1])
    pz = jnp.asarray(pos[:, 2])
    zeros = jnp.zeros((NPAD, DH), jnp.float32)

    w = _w_kernel(px, py, pz, srcf, dstf)
    wr = w.reshape(NS * NSTRIP, STRIP, CHUNK)
    wr = jnp.pad(wr, ((0, 0), (0, 0), (0, WPAD - CHUNK)))
    sd = _pack(src, dst)

    p0 = _seg_unweighted(x.reshape(2 * N, DH), sd, zeros)
    q0 = _seg_weighted(x.reshape(2 * N, DH), sd, wr, zeros)
    h, g = _mlp2(x, p0, W1a[0], b1a[0][None, :], W2a[0], b2a[0][None, :],
                 x, q0, W1b[0], b1b[0][None, :], W2b[0], b2b[0][None, :],
                 relu_out=True, normalize=False)
    p1 = _seg_unweighted(h.reshape(2 * N, DH), sd, zeros)
    q1 = _seg_weighted(g.reshape(2 * N, DH), sd, wr, zeros)
    z1, z2 = _mlp2(h, p1, W1a[1], b1a[1][None, :], W2a[1], b2a[1][None, :],
                   g, q1, W1b[1], b1b[1][None, :], W2b[1], b2b[1][None, :],
                   relu_out=False, normalize=True)

    return (z1, z2)


# speedup vs baseline: 1.0483x; 1.0483x over previous
import functools

import jax
import jax.numpy as jnp
from jax import lax
from jax.experimental import pallas as pl
from jax.experimental.pallas import tpu as pltpu
from jax.experimental.pallas import tpu_sc as plsc

N = 10000
E = 320000
D = 128

NC = 2
NS = 16
NW = NC * NS
EPW = E // NW
EPS = E // NS
DH = D // NC
CHUNK = 125
STRIP = 8
NSTRIP = EPS // (STRIP * CHUNK)
NPAD = 10112
RPW = NPAD // NS
WPAD = 128
G16 = EPW // 16

_MESH = plsc.VectorSubcoreMesh(core_axis_name="c", subcore_axis_name="s")


def _seg_body(weighted, *refs):
    if weighted:
        (h_hbm, sdw_hbm, w_hbm, zeros_hbm, out_hbm,
         sdw0, sdw1, wv0, wv1, r0, r1, r2, r3,
         sg0, sg1, sg2, sg3, ss0, ss1, ss2, ss3, si0, si1, acc) = refs
        wvb = (wv0, wv1)
    else:
        (h_hbm, sdw_hbm, zeros_hbm, out_hbm,
         sdw0, sdw1, r0, r1, r2, r3,
         sg0, sg1, sg2, sg3, ss0, ss1, ss2, ss3, si0, si1, acc) = refs
        w_hbm = None
        wvb = (None, None)
    sdwb = (sdw0, sdw1)
    rb = (r0, r1, r2, r3)
    sgb = (sg0, sg1, sg2, sg3)
    ssb = (ss0, ss1, ss2, ss3)
    sib = (si0, si1)

    c = lax.axis_index("c")
    s = lax.axis_index("s")

    def load_strip(t, p, sync):
        if sync:
            pltpu.sync_copy(sdw_hbm.at[c, s * NSTRIP + t], sdwb[p])
            if weighted:
                pltpu.sync_copy(w_hbm.at[s * NSTRIP + t], wvb[p])
        else:
            pltpu.async_copy(sdw_hbm.at[c, s * NSTRIP + t], sdwb[p], sib[p])
            if weighted:
                pltpu.async_copy(w_hbm.at[s * NSTRIP + t], wvb[p], sib[p])

    def wait_strip(p):
        pltpu.make_async_copy(sdw_hbm.at[0, 0], sdwb[p], sib[p]).wait()
        if weighted:
            pltpu.make_async_copy(w_hbm.at[0], wvb[p], sib[p]).wait()

    def fire_gather(tp, q, b):
        pltpu.async_copy(h_hbm.at[sdwb[tp].at[0, q]], rb[b], sgb[b])

    def wait_gather(b):
        pltpu.make_async_copy(h_hbm.at[sdwb[0].at[0, 0]], rb[b],
                              sgb[b]).wait()

    def fire_scatter(tp, q, b):
        pltpu.async_copy(rb[b], acc.at[sdwb[tp].at[1, q]], ssb[b], add=True)

    def wait_scatter(b):
        pltpu.make_async_copy(rb[b], acc.at[sdwb[0].at[1, 0]],
                              ssb[b]).wait()

    def scale(tp, q, b):
        if not weighted:
            return
        rows = rb[b]
        wvt = wvb[tp]
        qf = jnp.full((16,), q, jnp.int32)

        def scale_g16(g, carry2):
            base = g * 16
            wv16 = wvt[q, pl.ds(base, 16)]
            for lane in range(16):
                ws = wv16[lane]
                j = base + lane
                for k in range(DH // 16):
                    sl = (j, pl.ds(k * 16, 16))
                    rows[sl] = rows[sl] * ws
            return carry2

        lax.fori_loop(0, CHUNK // 16, scale_g16, 0)
        rbase = (CHUNK // 16) * 16
        wv16 = wvt[q, pl.ds(rbase, 16)]
        for lane in range(CHUNK - rbase):
            ws = wv16[lane]
            j = rbase + lane
            for k in range(DH // 16):
                sl = (j, pl.ds(k * 16, 16))
                rows[sl] = rows[sl] * ws

    def run_strip(t, tp, first, last):
        if first:
            fire_gather(tp, 0, 0)
            fire_gather(tp, 1, 1)
            fire_gather(tp, 2, 2)
        for q in range(STRIP):
            b = q % 4
            wait_gather(b)
            scale(tp, q, b)
            fire_scatter(tp, q, b)
            if not last and q == 1:
                load_strip(t + 1, 1 - tp, sync=False)
            nq = q + 3
            if nq < STRIP:
                if first and q == 0:
                    fire_gather(tp, nq, nq % 4)
                else:
                    wait_scatter(nq % 4)
                    fire_gather(tp, nq, nq % 4)
            elif not last:
                if q == STRIP - 3:
                    wait_strip(1 - tp)
                wait_scatter(nq % 4)
                fire_gather(1 - tp, nq - STRIP, nq % 4)

    pltpu.sync_copy(zeros_hbm.at[pl.ds(s * RPW, RPW)],
                    acc.at[pl.ds(s * RPW, RPW)])
    load_strip(0, 0, sync=True)
    plsc.subcore_barrier()

    run_strip(0, 0, first=True, last=False)

    def strip_pair(u, carry):
        t1 = 2 * u + 1
        run_strip(t1, 1, first=False, last=False)
        run_strip(t1 + 1, 0, first=False, last=False)
        return carry

    lax.fori_loop(0, (NSTRIP - 2) // 2, strip_pair, 0)

    run_strip(NSTRIP - 1, 1, first=False, last=True)

    for b in range(4):
        wait_scatter(b)
    plsc.subcore_barrier()
    pltpu.sync_copy(acc.at[pl.ds(s * RPW, RPW)],
                    out_hbm.at[c, pl.ds(s * RPW, RPW)])


def _make_seg(weighted):
    wscratch = [
        pltpu.VMEM((STRIP, WPAD), jnp.float32),
        pltpu.VMEM((STRIP, WPAD), jnp.float32),
    ] if weighted else []
    return pl.kernel(
        functools.partial(_seg_body, weighted),
        out_type=jax.ShapeDtypeStruct((NC, NPAD, DH), jnp.float32),
        mesh=_MESH,
        compiler_params=pltpu.CompilerParams(
            needs_layout_passes=False, use_tc_tiling_on_sc=False),
        scratch_types=[
            pltpu.VMEM((2, STRIP, CHUNK), jnp.int32),
            pltpu.VMEM((2, STRIP, CHUNK), jnp.int32),
        ] + wscratch + [
            pltpu.VMEM((CHUNK, DH), jnp.float32),
            pltpu.VMEM((CHUNK, DH), jnp.float32),
            pltpu.VMEM((CHUNK, DH), jnp.float32),
            pltpu.VMEM((CHUNK, DH), jnp.float32),
            pltpu.SemaphoreType.DMA,
            pltpu.SemaphoreType.DMA,
            pltpu.SemaphoreType.DMA,
            pltpu.SemaphoreType.DMA,
            pltpu.SemaphoreType.DMA,
            pltpu.SemaphoreType.DMA,
            pltpu.SemaphoreType.DMA,
            pltpu.SemaphoreType.DMA,
            pltpu.SemaphoreType.DMA,
            pltpu.SemaphoreType.DMA,
            pltpu.VMEM_SHARED((NPAD, DH), jnp.float32),
        ],
    )


_seg_unweighted = _make_seg(False)
_seg_weighted = _make_seg(True)


def _w_body(px_hbm, py_hbm, pz_hbm, src_hbm, dst_hbm, w_out,
            pxv, pyv, pzv, srcv, dstv, wv):
    c = lax.axis_index("c")
    s = lax.axis_index("s")
    wid = s * NC + c

    pltpu.sync_copy(px_hbm, pxv)
    pltpu.sync_copy(py_hbm, pyv)
    pltpu.sync_copy(pz_hbm, pzv)
    pltpu.sync_copy(src_hbm.at[wid], srcv)
    pltpu.sync_copy(dst_hbm.at[wid], dstv)

    def group(g, carry):
        sl = pl.ds(g * 16, 16)
        s16 = srcv[sl]
        d16 = dstv[sl]
        ddx = plsc.load_gather(pxv, [s16]) - plsc.load_gather(pxv, [d16])
        ddy = plsc.load_gather(pyv, [s16]) - plsc.load_gather(pyv, [d16])
        ddz = plsc.load_gather(pzv, [s16]) - plsc.load_gather(pzv, [d16])
        d2 = ddx * ddx + ddy * ddy + ddz * ddz
        wv[sl] = jnp.exp(-d2)
        return carry

    lax.fori_loop(0, G16, group, 0)
    pltpu.sync_copy(wv, w_out.at[wid])


_w_kernel = pl.kernel(
    _w_body,
    out_type=jax.ShapeDtypeStruct((NW, EPW), jnp.float32),
    mesh=_MESH,
    compiler_params=pltpu.CompilerParams(needs_layout_passes=False),
    scratch_types=[
        pltpu.VMEM((N,), jnp.float32),
        pltpu.VMEM((N,), jnp.float32),
        pltpu.VMEM((N,), jnp.float32),
        pltpu.VMEM((EPW,), jnp.int32),
        pltpu.VMEM((EPW,), jnp.int32),
        pltpu.VMEM((EPW,), jnp.float32),
    ],
)


def _mlp_one(h, p, w1, b1, w2, b2, relu_out, normalize):
    t = h + jnp.concatenate([p[0, :N, :], p[1, :N, :]], axis=1)
    u = jnp.maximum(
        jnp.dot(t, w1, preferred_element_type=jnp.float32) + b1, 0.0)
    v = jnp.dot(u, w2, preferred_element_type=jnp.float32) + b2
    if relu_out:
        v = jnp.maximum(v, 0.0)
    if normalize:
        mu = jnp.mean(v, axis=0, keepdims=True)
        var = jnp.sum((v - mu) * (v - mu), axis=0, keepdims=True) / (N - 1)
        v = (v - mu) * lax.rsqrt(var)
    return v


def _mlp_body(h_ref, p_ref, w1_ref, b1_ref, w2_ref, b2_ref, o_ref,
              *, relu_out, normalize):
    o_ref[...] = _mlp_one(h_ref[...], p_ref[...], w1_ref[...], b1_ref[...],
                          w2_ref[...], b2_ref[...], relu_out, normalize)


def _mlp(h, p, w1, b1, w2, b2, relu_out, normalize):
    return pl.pallas_call(
        functools.partial(_mlp_body, relu_out=relu_out, normalize=normalize),
        out_shape=jax.ShapeDtypeStruct((N, D), jnp.float32),
    )(h, p, w1, b1, w2, b2)


def _pack(src, dst):
    out = []
    for cc in range(NC):
        a = jnp.stack([2 * src + cc, dst])
        a = a.reshape(2, NS * NSTRIP, STRIP, CHUNK)
        out.append(jnp.transpose(a, (1, 0, 2, 3)))
    return jnp.stack(out)


def kernel(x, edge_index, pos, W1a, b1a, W2a, b2a, W1b, b1b, W2b, b2b):
    src = edge_index[0]
    dst = edge_index[1]
    srcf = src.reshape(NW, EPW)
    dstf = dst.reshape(NW, EPW)
    px = jnp.asarray(pos[:, 0])
    py = jnp.asarray(pos[:, 1])
    pz = jnp.asarray(pos[:, 2])
    zeros = jnp.zeros((NPAD, DH), jnp.float32)

    w = _w_kernel(px, py, pz, srcf, dstf)
    wr = w.reshape(NS * NSTRIP, STRIP, CHUNK)
    wr = jnp.pad(wr, ((0, 0), (0, 0), (0, WPAD - CHUNK)))
    sd = _pack(src, dst)

    p0 = _seg_unweighted(x.reshape(2 * N, DH), sd, zeros)
    q0 = _seg_weighted(x.reshape(2 * N, DH), sd, wr, zeros)
    h = _mlp(x, p0, W1a[0], b1a[0][None, :], W2a[0], b2a[0][None, :],
             relu_out=True, normalize=False)
    g = _mlp(x, q0, W1b[0], b1b[0][None, :], W2b[0], b2b[0][None, :],
             relu_out=True, normalize=False)
    p1 = _seg_unweighted(h.reshape(2 * N, DH), sd, zeros)
    q1 = _seg_weighted(g.reshape(2 * N, DH), sd, wr, zeros)
    z1 = _mlp(h, p1, W1a[1], b1a[1][None, :], W2a[1], b2a[1][None, :],
              relu_out=False, normalize=True)
    z2 = _mlp(g, q1, W1b[1], b1b[1][None, :], W2b[1], b2b[1][None, :],
              relu_out=False, normalize=True)

    return (z1, z2)
